# Initial kernel scaffold; baseline (speedup 1.0000x reference)
import jax
import jax.numpy as jnp
from jax.experimental import pallas as pl


def kernel(x1, x2, w1, b1, w2, b2):
    raise NotImplementedError("write your pallas kernel here")


# speedup vs baseline: 2.5628x; 2.5628x over previous
import math

import numpy as np

import jax
import jax.numpy as jnp
from jax.experimental import pallas as pl
from jax.experimental.pallas import tpu as pltpu

_VMEM_LIMIT = 64 * 1024 * 1024


def _interp_mat(n_in, n_out):
    m = np.zeros((n_out, n_in), np.float32)
    for i in range(n_out):
        src = 0.0 if n_out == 1 else i * (n_in - 1) / (n_out - 1)
        i0 = min(int(math.floor(src)), n_in - 1)
        i1 = min(i0 + 1, n_in - 1)
        f = src - i0
        m[i, i0] += 1.0 - f
        m[i, i1] += f
    return m


def _make_up_conv_body(Din, Dout, Hin, Win, H, W, Cu, Cs, Cm):
    HW = H * W
    Cin = Cu + Cs

    def body(x1_ref, x2m_ref, x2c_ref, x2p_ref, mhw_ref, w_ref, b_ref,
             o_ref, pad_ref, acc_ref):
        d = pl.program_id(1)
        nd = pl.num_programs(1)
        acc_ref[...] = jnp.zeros(acc_ref.shape, jnp.float32)
        pad_ref[...] = jnp.zeros(pad_ref.shape, pad_ref.dtype)

        def tap(kd, x2_ref):
            od = jnp.clip(d + (kd - 1), 0, Dout - 1)
            t = od * (Din - 1)
            i0 = t // (Dout - 1)
            i1 = jnp.minimum(i0 + 1, Din - 1)
            fd = (t % (Dout - 1)).astype(jnp.float32) * (1.0 / (Dout - 1))
            a0 = x1_ref[0, pl.ds(i0, 1)].reshape(Hin * Win, Cu)
            a0 = a0.astype(jnp.float32)
            a1 = x1_ref[0, pl.ds(i1, 1)].reshape(Hin * Win, Cu)
            a1 = a1.astype(jnp.float32)
            xz = (a0 + fd * (a1 - a0)).astype(jnp.bfloat16)
            u = jnp.dot(mhw_ref[...], xz, preferred_element_type=jnp.float32)
            pad_ref[1:H + 1, 1:W + 1, :Cu] = (
                u.reshape(H, W, Cu).astype(pad_ref.dtype))
            pad_ref[1:H + 1, 1:W + 1, Cu:] = x2_ref[...].reshape(H, W, Cs)
            for kh in range(3):
                for kw in range(3):
                    patch = pad_ref[kh:kh + H, kw:kw + W, :].reshape(HW, Cin)
                    acc_ref[...] += jnp.dot(
                        patch, w_ref[kd * 9 + kh * 3 + kw],
                        preferred_element_type=jnp.float32)

        tap(1, x2c_ref)

        @pl.when(d > 0)
        def _():
            tap(0, x2m_ref)

        @pl.when(d < nd - 1)
        def _():
            tap(2, x2p_ref)

        y = jnp.maximum(acc_ref[...] + b_ref[...], 0.0)
        o_ref[...] = y.reshape(1, 1, H, W, Cm).astype(o_ref.dtype)

    return body


def _up_conv1(x1b, x2b, w1t, b1, mhw):
    N, Din, Hin, Win, Cu = x1b.shape
    _, Dout, H, W, Cs = x2b.shape
    Cm = w1t.shape[-1]

    def sm(kd):
        off = kd - 1
        return lambda n, d: (n, jnp.clip(d + off, 0, Dout - 1), 0, 0, 0)

    return pl.pallas_call(
        _make_up_conv_body(Din, Dout, Hin, Win, H, W, Cu, Cs, Cm),
        out_shape=jax.ShapeDtypeStruct((N, Dout, H, W, Cm), jnp.bfloat16),
        grid=(N, Dout),
        in_specs=[
            pl.BlockSpec((1, Din, Hin, Win, Cu), lambda n, d: (n, 0, 0, 0, 0)),
            pl.BlockSpec((1, 1, H, W, Cs), sm(0)),
            pl.BlockSpec((1, 1, H, W, Cs), sm(1)),
            pl.BlockSpec((1, 1, H, W, Cs), sm(2)),
            pl.BlockSpec((H * W, Hin * Win), lambda n, d: (0, 0)),
            pl.BlockSpec(w1t.shape, lambda n, d: (0, 0, 0)),
            pl.BlockSpec((1, Cm), lambda n, d: (0, 0)),
        ],
        out_specs=pl.BlockSpec((1, 1, H, W, Cm),
                               lambda n, d: (n, d, 0, 0, 0)),
        scratch_shapes=[
            pltpu.VMEM((H + 2, W + 2, Cu + Cs), jnp.bfloat16),
            pltpu.VMEM((H * W, Cm), jnp.float32),
        ],
        compiler_params=pltpu.CompilerParams(
            dimension_semantics=("parallel", "parallel"),
            vmem_limit_bytes=_VMEM_LIMIT),
    )(x1b, x2b, x2b, x2b, mhw, w1t, b1.reshape(1, Cm))


def _make_conv_body(H, W, Cin, Cout):
    HW = H * W

    def body(xm_ref, xc_ref, xp_ref, w_ref, b_ref, o_ref, pad_ref, acc_ref):
        d = pl.program_id(1)
        nd = pl.num_programs(1)
        acc_ref[...] = jnp.zeros(acc_ref.shape, jnp.float32)
        pad_ref[...] = jnp.zeros(pad_ref.shape, pad_ref.dtype)

        def tap(kd, x_ref):
            pad_ref[1:H + 1, 1:W + 1, :] = x_ref[...].reshape(H, W, Cin)
            for kh in range(3):
                for kw in range(3):
                    patch = pad_ref[kh:kh + H, kw:kw + W, :].reshape(HW, Cin)
                    acc_ref[...] += jnp.dot(
                        patch, w_ref[kd * 9 + kh * 3 + kw],
                        preferred_element_type=jnp.float32)

        tap(1, xc_ref)

        @pl.when(d > 0)
        def _():
            tap(0, xm_ref)

        @pl.when(d < nd - 1)
        def _():
            tap(2, xp_ref)

        y = jnp.maximum(acc_ref[...] + b_ref[...], 0.0)
        o_ref[...] = y.reshape(1, 1, H, W, Cout).astype(o_ref.dtype)

    return body


def _conv2(h, w2t, b2, out_dtype):
    N, D, H, W, Cin = h.shape
    Cout = w2t.shape[-1]

    def sm(kd):
        off = kd - 1
        return lambda n, d: (n, jnp.clip(d + off, 0, D - 1), 0, 0, 0)

    return pl.pallas_call(
        _make_conv_body(H, W, Cin, Cout),
        out_shape=jax.ShapeDtypeStruct((N, D, H, W, Cout), out_dtype),
        grid=(N, D),
        in_specs=[
            pl.BlockSpec((1, 1, H, W, Cin), sm(0)),
            pl.BlockSpec((1, 1, H, W, Cin), sm(1)),
            pl.BlockSpec((1, 1, H, W, Cin), sm(2)),
            pl.BlockSpec(w2t.shape, lambda n, d: (0, 0, 0)),
            pl.BlockSpec((1, Cout), lambda n, d: (0, 0)),
        ],
        out_specs=pl.BlockSpec((1, 1, H, W, Cout),
                               lambda n, d: (n, d, 0, 0, 0)),
        scratch_shapes=[
            pltpu.VMEM((H + 2, W + 2, Cin), jnp.bfloat16),
            pltpu.VMEM((H * W, Cout), jnp.float32),
        ],
        compiler_params=pltpu.CompilerParams(
            dimension_semantics=("parallel", "parallel"),
            vmem_limit_bytes=_VMEM_LIMIT),
    )(h, h, h, w2t, b2.reshape(1, Cout))


def kernel(x1, x2, w1, b1, w2, b2):
    N, Cu, Din, Hin, Win = x1.shape
    Cs, Dout, H, W = x2.shape[1], x2.shape[2], x2.shape[3], x2.shape[4]
    Cm = w1.shape[0]

    x1b = jnp.transpose(x1, (0, 2, 3, 4, 1)).astype(jnp.bfloat16)
    x2b = jnp.transpose(x2, (0, 2, 3, 4, 1)).astype(jnp.bfloat16)
    w1t = jnp.transpose(w1, (2, 3, 4, 1, 0)).reshape(
        27, Cu + Cs, Cm).astype(jnp.bfloat16)
    w2t = jnp.transpose(w2, (2, 3, 4, 1, 0)).reshape(
        27, Cm, w2.shape[0]).astype(jnp.bfloat16)
    mhw = jnp.asarray(np.kron(_interp_mat(Hin, H), _interp_mat(Win, W)),
                      jnp.bfloat16)

    h = _up_conv1(x1b, x2b, w1t, b1, mhw)
    y = _conv2(h, w2t, b2, x1.dtype)
    return jnp.transpose(y, (0, 4, 1, 2, 3))


# speedup vs baseline: 2.6542x; 1.0357x over previous
import math

import numpy as np

import jax
import jax.numpy as jnp
from jax.experimental import pallas as pl
from jax.experimental.pallas import tpu as pltpu

_VMEM_LIMIT = 64 * 1024 * 1024


def _interp_mat(n_in, n_out):
    m = np.zeros((n_out, n_in), np.float32)
    for i in range(n_out):
        src = 0.0 if n_out == 1 else i * (n_in - 1) / (n_out - 1)
        i0 = min(int(math.floor(src)), n_in - 1)
        i1 = min(i0 + 1, n_in - 1)
        f = src - i0
        m[i, i0] += 1.0 - f
        m[i, i1] += f
    return m


def _wshift_variants(base, HW, W, C):
    wi = jax.lax.broadcasted_iota(jnp.int32, (HW, 1), 0) % W
    zrow = jnp.zeros((1, C), base.dtype)
    sm1 = jnp.where(wi >= 1,
                    jnp.concatenate([zrow, base[:HW - 1]], axis=0),
                    jnp.zeros_like(base))
    sp1 = jnp.where(wi <= W - 2,
                    jnp.concatenate([base[1:], zrow], axis=0),
                    jnp.zeros_like(base))
    return (sm1, base, sp1)


def _accum_taps(acc_ref, variants, w_ref, kd, HW, W):
    for kh in range(3):
        s = (kh - 1) * W
        for kw in range(3):
            y = jnp.dot(variants[kw], w_ref[kd * 9 + kh * 3 + kw],
                        preferred_element_type=jnp.float32)
            if s == 0:
                acc_ref[...] += y
            elif s > 0:
                acc_ref[0:HW - s, :] += y[s:, :]
            else:
                acc_ref[-s:HW, :] += y[0:HW + s, :]


def _make_up_conv_body(Din, Dout, Hin, Win, H, W, Cu, Cs, Cm):
    HW = H * W
    Cin = Cu + Cs

    def body(x1_ref, x2m_ref, x2c_ref, x2p_ref, mhw_ref, w_ref, b_ref,
             o_ref, acc_ref):
        d = pl.program_id(1)
        nd = pl.num_programs(1)
        acc_ref[...] = jnp.zeros(acc_ref.shape, jnp.float32)

        def tap(kd, x2_ref):
            od = jnp.clip(d + (kd - 1), 0, Dout - 1)
            t = od * (Din - 1)
            i0 = t // (Dout - 1)
            i1 = jnp.minimum(i0 + 1, Din - 1)
            fd = (t % (Dout - 1)).astype(jnp.float32) * (1.0 / (Dout - 1))
            a0 = x1_ref[0, pl.ds(i0, 1)].reshape(Hin * Win, Cu)
            a0 = a0.astype(jnp.float32)
            a1 = x1_ref[0, pl.ds(i1, 1)].reshape(Hin * Win, Cu)
            a1 = a1.astype(jnp.float32)
            xz = (a0 + fd * (a1 - a0)).astype(jnp.bfloat16)
            u = jnp.dot(mhw_ref[...], xz, preferred_element_type=jnp.float32)
            base = jnp.concatenate(
                [u.astype(jnp.bfloat16), x2_ref[...].reshape(HW, Cs)], axis=1)
            variants = _wshift_variants(base, HW, W, Cin)
            _accum_taps(acc_ref, variants, w_ref, kd, HW, W)

        tap(1, x2c_ref)

        @pl.when(d > 0)
        def _():
            tap(0, x2m_ref)

        @pl.when(d < nd - 1)
        def _():
            tap(2, x2p_ref)

        y = jnp.maximum(acc_ref[...] + b_ref[...], 0.0)
        o_ref[...] = y.reshape(1, 1, H, W, Cm).astype(o_ref.dtype)

    return body


def _up_conv1(x1b, x2b, w1t, b1, mhw):
    N, Din, Hin, Win, Cu = x1b.shape
    _, Dout, H, W, Cs = x2b.shape
    Cm = w1t.shape[-1]

    def sm(kd):
        off = kd - 1
        return lambda n, d: (n, jnp.clip(d + off, 0, Dout - 1), 0, 0, 0)

    return pl.pallas_call(
        _make_up_conv_body(Din, Dout, Hin, Win, H, W, Cu, Cs, Cm),
        out_shape=jax.ShapeDtypeStruct((N, Dout, H, W, Cm), jnp.bfloat16),
        grid=(N, Dout),
        in_specs=[
            pl.BlockSpec((1, Din, Hin, Win, Cu), lambda n, d: (n, 0, 0, 0, 0)),
            pl.BlockSpec((1, 1, H, W, Cs), sm(0)),
            pl.BlockSpec((1, 1, H, W, Cs), sm(1)),
            pl.BlockSpec((1, 1, H, W, Cs), sm(2)),
            pl.BlockSpec((H * W, Hin * Win), lambda n, d: (0, 0)),
            pl.BlockSpec(w1t.shape, lambda n, d: (0, 0, 0)),
            pl.BlockSpec((1, Cm), lambda n, d: (0, 0)),
        ],
        out_specs=pl.BlockSpec((1, 1, H, W, Cm),
                               lambda n, d: (n, d, 0, 0, 0)),
        scratch_shapes=[
            pltpu.VMEM((H * W, Cm), jnp.float32),
        ],
        compiler_params=pltpu.CompilerParams(
            dimension_semantics=("parallel", "parallel"),
            vmem_limit_bytes=_VMEM_LIMIT),
    )(x1b, x2b, x2b, x2b, mhw, w1t, b1.reshape(1, Cm))


def _make_conv_body(H, W, Cin, Cout):
    HW = H * W

    def body(xm_ref, xc_ref, xp_ref, w_ref, b_ref, o_ref, acc_ref):
        d = pl.program_id(1)
        nd = pl.num_programs(1)
        acc_ref[...] = jnp.zeros(acc_ref.shape, jnp.float32)

        def tap(kd, x_ref):
            base = x_ref[...].reshape(HW, Cin)
            variants = _wshift_variants(base, HW, W, Cin)
            _accum_taps(acc_ref, variants, w_ref, kd, HW, W)

        tap(1, xc_ref)

        @pl.when(d > 0)
        def _():
            tap(0, xm_ref)

        @pl.when(d < nd - 1)
        def _():
            tap(2, xp_ref)

        y = jnp.maximum(acc_ref[...] + b_ref[...], 0.0)
        o_ref[...] = y.reshape(1, 1, H, W, Cout).astype(o_ref.dtype)

    return body


def _conv2(h, w2t, b2, out_dtype):
    N, D, H, W, Cin = h.shape
    Cout = w2t.shape[-1]

    def sm(kd):
        off = kd - 1
        return lambda n, d: (n, jnp.clip(d + off, 0, D - 1), 0, 0, 0)

    return pl.pallas_call(
        _make_conv_body(H, W, Cin, Cout),
        out_shape=jax.ShapeDtypeStruct((N, D, H, W, Cout), out_dtype),
        grid=(N, D),
        in_specs=[
            pl.BlockSpec((1, 1, H, W, Cin), sm(0)),
            pl.BlockSpec((1, 1, H, W, Cin), sm(1)),
            pl.BlockSpec((1, 1, H, W, Cin), sm(2)),
            pl.BlockSpec(w2t.shape, lambda n, d: (0, 0, 0)),
            pl.BlockSpec((1, Cout), lambda n, d: (0, 0)),
        ],
        out_specs=pl.BlockSpec((1, 1, H, W, Cout),
                               lambda n, d: (n, d, 0, 0, 0)),
        scratch_shapes=[
            pltpu.VMEM((H * W, Cout), jnp.float32),
        ],
        compiler_params=pltpu.CompilerParams(
            dimension_semantics=("parallel", "parallel"),
            vmem_limit_bytes=_VMEM_LIMIT),
    )(h, h, h, w2t, b2.reshape(1, Cout))


def kernel(x1, x2, w1, b1, w2, b2):
    N, Cu, Din, Hin, Win = x1.shape
    Cs, Dout, H, W = x2.shape[1], x2.shape[2], x2.shape[3], x2.shape[4]
    Cm = w1.shape[0]

    x1b = jnp.transpose(x1, (0, 2, 3, 4, 1)).astype(jnp.bfloat16)
    x2b = jnp.transpose(x2, (0, 2, 3, 4, 1)).astype(jnp.bfloat16)
    w1t = jnp.transpose(w1, (2, 3, 4, 1, 0)).reshape(
        27, Cu + Cs, Cm).astype(jnp.bfloat16)
    w2t = jnp.transpose(w2, (2, 3, 4, 1, 0)).reshape(
        27, Cm, w2.shape[0]).astype(jnp.bfloat16)
    mhw = jnp.asarray(np.kron(_interp_mat(Hin, H), _interp_mat(Win, W)),
                      jnp.bfloat16)

    h = _up_conv1(x1b, x2b, w1t, b1, mhw)
    y = _conv2(h, w2t, b2, x1.dtype)
    return jnp.transpose(y, (0, 4, 1, 2, 3))


# speedup vs baseline: 3.0373x; 1.1443x over previous
import math

import numpy as np

import jax
import jax.numpy as jnp
from jax.experimental import pallas as pl
from jax.experimental.pallas import tpu as pltpu

_VMEM_LIMIT = 64 * 1024 * 1024


def _interp_mat(n_in, n_out):
    m = np.zeros((n_out, n_in), np.float32)
    for i in range(n_out):
        src = 0.0 if n_out == 1 else i * (n_in - 1) / (n_out - 1)
        i0 = min(int(math.floor(src)), n_in - 1)
        i1 = min(i0 + 1, n_in - 1)
        f = src - i0
        m[i, i0] += 1.0 - f
        m[i, i1] += f
    return m


def _wshift_variants(base, HW, W, C):
    wi = jax.lax.broadcasted_iota(jnp.int32, (HW, 1), 0) % W
    zrow = jnp.zeros((1, C), base.dtype)
    sm1 = jnp.where(wi >= 1,
                    jnp.concatenate([zrow, base[:HW - 1]], axis=0),
                    jnp.zeros_like(base))
    sp1 = jnp.where(wi <= W - 2,
                    jnp.concatenate([base[1:], zrow], axis=0),
                    jnp.zeros_like(base))
    return (sm1, base, sp1)


def _accum_taps(acc_ref, variants, w_ref, kd, HW, W):
    for kh in range(3):
        s = (kh - 1) * W
        y = None
        for kw in range(3):
            p = jnp.dot(variants[kw], w_ref[kd * 9 + kh * 3 + kw],
                        preferred_element_type=jnp.float32)
            y = p if y is None else y + p
        if s == 0:
            acc_ref[...] += y
        elif s > 0:
            acc_ref[0:HW - s, :] += y[s:, :]
        else:
            acc_ref[-s:HW, :] += y[0:HW + s, :]


def _make_up_conv_body(Din, Dout, Hin, Win, H, W, Cu, Cs, Cm):
    HW = H * W
    Cin = Cu + Cs

    def body(x1_ref, x2m_ref, x2c_ref, x2p_ref, mhw_ref, w_ref, b_ref,
             o_ref, acc_ref):
        d = pl.program_id(1)
        nd = pl.num_programs(1)
        acc_ref[...] = jnp.zeros(acc_ref.shape, jnp.float32)

        def tap(kd, x2_ref):
            od = jnp.clip(d + (kd - 1), 0, Dout - 1)
            t = od * (Din - 1)
            i0 = t // (Dout - 1)
            i1 = jnp.minimum(i0 + 1, Din - 1)
            fd = (t % (Dout - 1)).astype(jnp.float32) * (1.0 / (Dout - 1))
            a0 = x1_ref[0, pl.ds(i0, 1)].reshape(Hin * Win, Cu)
            a0 = a0.astype(jnp.float32)
            a1 = x1_ref[0, pl.ds(i1, 1)].reshape(Hin * Win, Cu)
            a1 = a1.astype(jnp.float32)
            xz = (a0 + fd * (a1 - a0)).astype(jnp.bfloat16)
            u = jnp.dot(mhw_ref[...], xz, preferred_element_type=jnp.float32)
            base = jnp.concatenate(
                [u.astype(jnp.bfloat16), x2_ref[...].reshape(HW, Cs)], axis=1)
            variants = _wshift_variants(base, HW, W, Cin)
            _accum_taps(acc_ref, variants, w_ref, kd, HW, W)

        tap(1, x2c_ref)

        @pl.when(d > 0)
        def _():
            tap(0, x2m_ref)

        @pl.when(d < nd - 1)
        def _():
            tap(2, x2p_ref)

        y = jnp.maximum(acc_ref[...] + b_ref[...], 0.0)
        o_ref[...] = y.reshape(1, 1, H, W, Cm).astype(o_ref.dtype)

    return body


def _up_conv1(x1b, x2b, w1t, b1, mhw):
    N, Din, Hin, Win, Cu = x1b.shape
    _, Dout, H, W, Cs = x2b.shape
    Cm = w1t.shape[-1]

    def sm(kd):
        off = kd - 1
        return lambda n, d: (n, jnp.clip(d + off, 0, Dout - 1), 0, 0, 0)

    return pl.pallas_call(
        _make_up_conv_body(Din, Dout, Hin, Win, H, W, Cu, Cs, Cm),
        out_shape=jax.ShapeDtypeStruct((N, Dout, H, W, Cm), jnp.bfloat16),
        grid=(N, Dout),
        in_specs=[
            pl.BlockSpec((1, Din, Hin, Win, Cu), lambda n, d: (n, 0, 0, 0, 0)),
            pl.BlockSpec((1, 1, H, W, Cs), sm(0)),
            pl.BlockSpec((1, 1, H, W, Cs), sm(1)),
            pl.BlockSpec((1, 1, H, W, Cs), sm(2)),
            pl.BlockSpec((H * W, Hin * Win), lambda n, d: (0, 0)),
            pl.BlockSpec(w1t.shape, lambda n, d: (0, 0, 0)),
            pl.BlockSpec((1, Cm), lambda n, d: (0, 0)),
        ],
        out_specs=pl.BlockSpec((1, 1, H, W, Cm),
                               lambda n, d: (n, d, 0, 0, 0)),
        scratch_shapes=[
            pltpu.VMEM((H * W, Cm), jnp.float32),
        ],
        compiler_params=pltpu.CompilerParams(
            dimension_semantics=("parallel", "parallel"),
            vmem_limit_bytes=_VMEM_LIMIT),
    )(x1b, x2b, x2b, x2b, mhw, w1t, b1.reshape(1, Cm))


def _make_conv_body(H, W, Cin, Cout):
    HW = H * W

    def body(xm_ref, xc_ref, xp_ref, w_ref, b_ref, o_ref, acc_ref):
        d = pl.program_id(1)
        nd = pl.num_programs(1)
        acc_ref[...] = jnp.zeros(acc_ref.shape, jnp.float32)

        def tap(kd, x_ref):
            base = x_ref[...].reshape(HW, Cin)
            variants = _wshift_variants(base, HW, W, Cin)
            _accum_taps(acc_ref, variants, w_ref, kd, HW, W)

        tap(1, xc_ref)

        @pl.when(d > 0)
        def _():
            tap(0, xm_ref)

        @pl.when(d < nd - 1)
        def _():
            tap(2, xp_ref)

        y = jnp.maximum(acc_ref[...] + b_ref[...], 0.0)
        o_ref[...] = y.reshape(1, 1, H, W, Cout).astype(o_ref.dtype)

    return body


def _conv2(h, w2t, b2, out_dtype):
    N, D, H, W, Cin = h.shape
    Cout = w2t.shape[-1]

    def sm(kd):
        off = kd - 1
        return lambda n, d: (n, jnp.clip(d + off, 0, D - 1), 0, 0, 0)

    return pl.pallas_call(
        _make_conv_body(H, W, Cin, Cout),
        out_shape=jax.ShapeDtypeStruct((N, D, H, W, Cout), out_dtype),
        grid=(N, D),
        in_specs=[
            pl.BlockSpec((1, 1, H, W, Cin), sm(0)),
            pl.BlockSpec((1, 1, H, W, Cin), sm(1)),
            pl.BlockSpec((1, 1, H, W, Cin), sm(2)),
            pl.BlockSpec(w2t.shape, lambda n, d: (0, 0, 0)),
            pl.BlockSpec((1, Cout), lambda n, d: (0, 0)),
        ],
        out_specs=pl.BlockSpec((1, 1, H, W, Cout),
                               lambda n, d: (n, d, 0, 0, 0)),
        scratch_shapes=[
            pltpu.VMEM((H * W, Cout), jnp.float32),
        ],
        compiler_params=pltpu.CompilerParams(
            dimension_semantics=("parallel", "parallel"),
            vmem_limit_bytes=_VMEM_LIMIT),
    )(h, h, h, w2t, b2.reshape(1, Cout))


def kernel(x1, x2, w1, b1, w2, b2):
    N, Cu, Din, Hin, Win = x1.shape
    Cs, Dout, H, W = x2.shape[1], x2.shape[2], x2.shape[3], x2.shape[4]
    Cm = w1.shape[0]

    x1b = jnp.transpose(x1, (0, 2, 3, 4, 1)).astype(jnp.bfloat16)
    x2b = jnp.transpose(x2, (0, 2, 3, 4, 1)).astype(jnp.bfloat16)
    w1t = jnp.transpose(w1, (2, 3, 4, 1, 0)).reshape(
        27, Cu + Cs, Cm).astype(jnp.bfloat16)
    w2t = jnp.transpose(w2, (2, 3, 4, 1, 0)).reshape(
        27, Cm, w2.shape[0]).astype(jnp.bfloat16)
    mhw = jnp.asarray(np.kron(_interp_mat(Hin, H), _interp_mat(Win, W)),
                      jnp.bfloat16)

    h = _up_conv1(x1b, x2b, w1t, b1, mhw)
    y = _conv2(h, w2t, b2, x1.dtype)
    return jnp.transpose(y, (0, 4, 1, 2, 3))


# speedup vs baseline: 3.2348x; 1.0650x over previous
import math

import numpy as np

import jax
import jax.numpy as jnp
from jax.experimental import pallas as pl
from jax.experimental.pallas import tpu as pltpu

_VMEM_LIMIT = 64 * 1024 * 1024
_DB = 2


def _interp_mat(n_in, n_out):
    m = np.zeros((n_out, n_in), np.float32)
    for i in range(n_out):
        src = 0.0 if n_out == 1 else i * (n_in - 1) / (n_out - 1)
        i0 = min(int(math.floor(src)), n_in - 1)
        i1 = min(i0 + 1, n_in - 1)
        f = src - i0
        m[i, i0] += 1.0 - f
        m[i, i1] += f
    return m


def _wshift_variants(base, HW, W, C):
    wi = jax.lax.broadcasted_iota(jnp.int32, (HW, 1), 0) % W
    zrow = jnp.zeros((1, C), base.dtype)
    sm1 = jnp.where(wi >= 1,
                    jnp.concatenate([zrow, base[:HW - 1]], axis=0),
                    jnp.zeros_like(base))
    sp1 = jnp.where(wi <= W - 2,
                    jnp.concatenate([base[1:], zrow], axis=0),
                    jnp.zeros_like(base))
    return (sm1, base, sp1)


def _accum_taps(acc_ref, k, variants, w_ref, kd, HW, W):
    for kh in range(3):
        s = (kh - 1) * W
        y = None
        for kw in range(3):
            p = jnp.dot(variants[kw], w_ref[kd * 9 + kh * 3 + kw],
                        preferred_element_type=jnp.float32)
            y = p if y is None else y + p
        if s == 0:
            acc_ref[k] += y
        elif s > 0:
            acc_ref[k, 0:HW - s, :] += y[s:, :]
        else:
            acc_ref[k, -s:HW, :] += y[0:HW + s, :]


def _chunk_conv(acc_ref, var_list, w_ref, c, nc, HW, W, DB):
    for k in range(DB):
        for kd in range(3):
            j = k + kd

            def do(k=k, kd=kd, j=j):
                _accum_taps(acc_ref, k, var_list[j], w_ref, kd, HW, W)

            if k == 0 and kd == 0:
                pl.when(c > 0)(do)
            elif k == DB - 1 and kd == 2:
                pl.when(c < nc - 1)(do)
            else:
                do()


def _make_up_conv_body(Din, Dout, Hin, Win, H, W, Cu, Cs, Cm, DB):
    HW = H * W
    Cin = Cu + Cs

    def body(*refs):
        x1_ref = refs[0]
        x2_refs = refs[1:DB + 3]
        mhw_ref, w_ref, b_ref, o_ref, acc_ref = refs[DB + 3:]
        c = pl.program_id(1)
        nc = pl.num_programs(1)
        acc_ref[...] = jnp.zeros(acc_ref.shape, jnp.float32)

        var_list = []
        for j in range(DB + 2):
            od = jnp.clip(c * DB + (j - 1), 0, Dout - 1)
            t = od * (Din - 1)
            i0 = t // (Dout - 1)
            i1 = jnp.minimum(i0 + 1, Din - 1)
            fd = (t % (Dout - 1)).astype(jnp.float32) * (1.0 / (Dout - 1))
            a0 = x1_ref[0, pl.ds(i0, 1)].reshape(Hin * Win, Cu)
            a1 = x1_ref[0, pl.ds(i1, 1)].reshape(Hin * Win, Cu)
            xz = (a0.astype(jnp.float32)
                  + fd * (a1.astype(jnp.float32) - a0.astype(jnp.float32)))
            u = jnp.dot(mhw_ref[...], xz.astype(jnp.bfloat16),
                        preferred_element_type=jnp.float32)
            base = jnp.concatenate(
                [u.astype(jnp.bfloat16), x2_refs[j][...].reshape(HW, Cs)],
                axis=1)
            var_list.append(_wshift_variants(base, HW, W, Cin))

        _chunk_conv(acc_ref, var_list, w_ref, c, nc, HW, W, DB)

        y = jnp.maximum(acc_ref[...] + b_ref[...], 0.0)
        o_ref[...] = y.reshape(1, DB, H, W, Cm).astype(o_ref.dtype)

    return body


def _up_conv1(x1b, x2b, w1t, b1, mhw, DB):
    N, Din, Hin, Win, Cu = x1b.shape
    _, Dout, H, W, Cs = x2b.shape
    Cm = w1t.shape[-1]
    nc = Dout // DB

    def sm(j):
        return lambda n, c: (n, jnp.clip(c * DB + (j - 1), 0, Dout - 1),
                             0, 0, 0)

    in_specs = [pl.BlockSpec((1, Din, Hin, Win, Cu),
                             lambda n, c: (n, 0, 0, 0, 0))]
    args = [x1b]
    for j in range(DB + 2):
        in_specs.append(pl.BlockSpec((1, 1, H, W, Cs), sm(j)))
        args.append(x2b)
    in_specs += [
        pl.BlockSpec((H * W, Hin * Win), lambda n, c: (0, 0)),
        pl.BlockSpec(w1t.shape, lambda n, c: (0, 0, 0)),
        pl.BlockSpec((1, Cm), lambda n, c: (0, 0)),
    ]
    args += [mhw, w1t, b1.reshape(1, Cm)]

    return pl.pallas_call(
        _make_up_conv_body(Din, Dout, Hin, Win, H, W, Cu, Cs, Cm, DB),
        out_shape=jax.ShapeDtypeStruct((N, Dout, H, W, Cm), jnp.bfloat16),
        grid=(N, nc),
        in_specs=in_specs,
        out_specs=pl.BlockSpec((1, DB, H, W, Cm),
                               lambda n, c: (n, c, 0, 0, 0)),
        scratch_shapes=[
            pltpu.VMEM((DB, H * W, Cm), jnp.float32),
        ],
        compiler_params=pltpu.CompilerParams(
            dimension_semantics=("parallel", "parallel"),
            vmem_limit_bytes=_VMEM_LIMIT),
    )(*args)


def _make_conv_body(H, W, Cin, Cout, DB):
    HW = H * W

    def body(*refs):
        x_refs = refs[:DB + 2]
        w_ref, b_ref, o_ref, acc_ref = refs[DB + 2:]
        c = pl.program_id(1)
        nc = pl.num_programs(1)
        acc_ref[...] = jnp.zeros(acc_ref.shape, jnp.float32)

        var_list = [
            _wshift_variants(x_refs[j][...].reshape(HW, Cin), HW, W, Cin)
            for j in range(DB + 2)]

        _chunk_conv(acc_ref, var_list, w_ref, c, nc, HW, W, DB)

        y = jnp.maximum(acc_ref[...] + b_ref[...], 0.0)
        o_ref[...] = y.reshape(1, DB, H, W, Cout).astype(o_ref.dtype)

    return body


def _conv2(h, w2t, b2, out_dtype, DB):
    N, D, H, W, Cin = h.shape
    Cout = w2t.shape[-1]
    nc = D // DB

    def sm(j):
        return lambda n, c: (n, jnp.clip(c * DB + (j - 1), 0, D - 1), 0, 0, 0)

    in_specs = [pl.BlockSpec((1, 1, H, W, Cin), sm(j)) for j in range(DB + 2)]
    args = [h] * (DB + 2)
    in_specs += [
        pl.BlockSpec(w2t.shape, lambda n, c: (0, 0, 0)),
        pl.BlockSpec((1, Cout), lambda n, c: (0, 0)),
    ]
    args += [w2t, b2.reshape(1, Cout)]

    return pl.pallas_call(
        _make_conv_body(H, W, Cin, Cout, DB),
        out_shape=jax.ShapeDtypeStruct((N, D, H, W, Cout), out_dtype),
        grid=(N, nc),
        in_specs=in_specs,
        out_specs=pl.BlockSpec((1, DB, H, W, Cout),
                               lambda n, c: (n, c, 0, 0, 0)),
        scratch_shapes=[
            pltpu.VMEM((DB, H * W, Cout), jnp.float32),
        ],
        compiler_params=pltpu.CompilerParams(
            dimension_semantics=("parallel", "parallel"),
            vmem_limit_bytes=_VMEM_LIMIT),
    )(*args)


def kernel(x1, x2, w1, b1, w2, b2):
    N, Cu, Din, Hin, Win = x1.shape
    Cs, Dout, H, W = x2.shape[1], x2.shape[2], x2.shape[3], x2.shape[4]
    Cm = w1.shape[0]
    db = _DB if Dout % _DB == 0 else 1

    x1b = jnp.transpose(x1, (0, 2, 3, 4, 1)).astype(jnp.bfloat16)
    x2b = jnp.transpose(x2, (0, 2, 3, 4, 1)).astype(jnp.bfloat16)
    w1t = jnp.transpose(w1, (2, 3, 4, 1, 0)).reshape(
        27, Cu + Cs, Cm).astype(jnp.bfloat16)
    w2t = jnp.transpose(w2, (2, 3, 4, 1, 0)).reshape(
        27, Cm, w2.shape[0]).astype(jnp.bfloat16)
    mhw = jnp.asarray(np.kron(_interp_mat(Hin, H), _interp_mat(Win, W)),
                      jnp.bfloat16)

    h = _up_conv1(x1b, x2b, w1t, b1, mhw, db)
    y = _conv2(h, w2t, b2, x1.dtype, db)
    return jnp.transpose(y, (0, 4, 1, 2, 3))


# speedup vs baseline: 3.5124x; 1.0858x over previous
import math

import numpy as np

import jax
import jax.numpy as jnp
from jax.experimental import pallas as pl
from jax.experimental.pallas import tpu as pltpu

_VMEM_LIMIT = 64 * 1024 * 1024
_DB = 4


def _interp_mat(n_in, n_out):
    m = np.zeros((n_out, n_in), np.float32)
    for i in range(n_out):
        src = 0.0 if n_out == 1 else i * (n_in - 1) / (n_out - 1)
        i0 = min(int(math.floor(src)), n_in - 1)
        i1 = min(i0 + 1, n_in - 1)
        f = src - i0
        m[i, i0] += 1.0 - f
        m[i, i1] += f
    return m


def _wshift_variants(base, HW, W, C):
    wi = jax.lax.broadcasted_iota(jnp.int32, (HW, 1), 0) % W
    zrow = jnp.zeros((1, C), base.dtype)
    sm1 = jnp.where(wi >= 1,
                    jnp.concatenate([zrow, base[:HW - 1]], axis=0),
                    jnp.zeros_like(base))
    sp1 = jnp.where(wi <= W - 2,
                    jnp.concatenate([base[1:], zrow], axis=0),
                    jnp.zeros_like(base))
    return (sm1, base, sp1)


def _accum_taps(acc_ref, k, variants, w_ref, kd, HW, W):
    for kh in range(3):
        s = (kh - 1) * W
        y = None
        for kw in range(3):
            p = jnp.dot(variants[kw], w_ref[kd * 9 + kh * 3 + kw],
                        preferred_element_type=jnp.float32)
            y = p if y is None else y + p
        if s == 0:
            acc_ref[k] += y
        elif s > 0:
            acc_ref[k, 0:HW - s, :] += y[s:, :]
        else:
            acc_ref[k, -s:HW, :] += y[0:HW + s, :]


def _chunk_conv(acc_ref, var_list, w_ref, c, nc, HW, W, DB):
    for k in range(DB):
        for kd in range(3):
            j = k + kd

            def do(k=k, kd=kd, j=j):
                _accum_taps(acc_ref, k, var_list[j], w_ref, kd, HW, W)

            if k == 0 and kd == 0:
                pl.when(c > 0)(do)
            elif k == DB - 1 and kd == 2:
                pl.when(c < nc - 1)(do)
            else:
                do()


def _make_up_conv_body(Din, Dout, Hin, Win, H, W, Cu, Cs, Cm, DB):
    HW = H * W
    Cin = Cu + Cs

    def body(*refs):
        x1_ref = refs[0]
        x2_refs = refs[1:DB + 3]
        mhw_ref, w_ref, b_ref, o_ref, acc_ref = refs[DB + 3:]
        c = pl.program_id(1)
        nc = pl.num_programs(1)
        acc_ref[...] = jnp.zeros(acc_ref.shape, jnp.float32)

        var_list = []
        for j in range(DB + 2):
            od = jnp.clip(c * DB + (j - 1), 0, Dout - 1)
            t = od * (Din - 1)
            i0 = t // (Dout - 1)
            i1 = jnp.minimum(i0 + 1, Din - 1)
            fd = (t % (Dout - 1)).astype(jnp.float32) * (1.0 / (Dout - 1))
            a0 = x1_ref[0, pl.ds(i0, 1)].reshape(Hin * Win, Cu)
            a1 = x1_ref[0, pl.ds(i1, 1)].reshape(Hin * Win, Cu)
            xz = (a0.astype(jnp.float32)
                  + fd * (a1.astype(jnp.float32) - a0.astype(jnp.float32)))
            u = jnp.dot(mhw_ref[...], xz.astype(jnp.bfloat16),
                        preferred_element_type=jnp.float32)
            base = jnp.concatenate(
                [u.astype(jnp.bfloat16), x2_refs[j][...].reshape(HW, Cs)],
                axis=1)
            var_list.append(_wshift_variants(base, HW, W, Cin))

        _chunk_conv(acc_ref, var_list, w_ref, c, nc, HW, W, DB)

        y = jnp.maximum(acc_ref[...] + b_ref[...], 0.0)
        o_ref[...] = y.reshape(1, DB, H, W, Cm).astype(o_ref.dtype)

    return body


def _up_conv1(x1b, x2b, w1t, b1, mhw, DB):
    N, Din, Hin, Win, Cu = x1b.shape
    _, Dout, H, W, Cs = x2b.shape
    Cm = w1t.shape[-1]
    nc = Dout // DB

    def sm(j):
        return lambda n, c: (n, jnp.clip(c * DB + (j - 1), 0, Dout - 1),
                             0, 0, 0)

    in_specs = [pl.BlockSpec((1, Din, Hin, Win, Cu),
                             lambda n, c: (n, 0, 0, 0, 0))]
    args = [x1b]
    for j in range(DB + 2):
        in_specs.append(pl.BlockSpec((1, 1, H, W, Cs), sm(j)))
        args.append(x2b)
    in_specs += [
        pl.BlockSpec((H * W, Hin * Win), lambda n, c: (0, 0)),
        pl.BlockSpec(w1t.shape, lambda n, c: (0, 0, 0)),
        pl.BlockSpec((1, Cm), lambda n, c: (0, 0)),
    ]
    args += [mhw, w1t, b1.reshape(1, Cm)]

    return pl.pallas_call(
        _make_up_conv_body(Din, Dout, Hin, Win, H, W, Cu, Cs, Cm, DB),
        out_shape=jax.ShapeDtypeStruct((N, Dout, H, W, Cm), jnp.bfloat16),
        grid=(N, nc),
        in_specs=in_specs,
        out_specs=pl.BlockSpec((1, DB, H, W, Cm),
                               lambda n, c: (n, c, 0, 0, 0)),
        scratch_shapes=[
            pltpu.VMEM((DB, H * W, Cm), jnp.float32),
        ],
        compiler_params=pltpu.CompilerParams(
            dimension_semantics=("parallel", "parallel"),
            vmem_limit_bytes=_VMEM_LIMIT),
    )(*args)


def _make_conv_body(H, W, Cin, Cout, DB):
    HW = H * W

    def body(*refs):
        x_refs = refs[:DB + 2]
        w_ref, b_ref, o_ref, acc_ref = refs[DB + 2:]
        c = pl.program_id(1)
        nc = pl.num_programs(1)
        acc_ref[...] = jnp.zeros(acc_ref.shape, jnp.float32)

        var_list = [
            _wshift_variants(x_refs[j][...].reshape(HW, Cin), HW, W, Cin)
            for j in range(DB + 2)]

        _chunk_conv(acc_ref, var_list, w_ref, c, nc, HW, W, DB)

        y = jnp.maximum(acc_ref[...] + b_ref[...], 0.0)
        o_ref[...] = y.reshape(1, DB, H, W, Cout).astype(o_ref.dtype)

    return body


def _conv2(h, w2t, b2, out_dtype, DB):
    N, D, H, W, Cin = h.shape
    Cout = w2t.shape[-1]
    nc = D // DB

    def sm(j):
        return lambda n, c: (n, jnp.clip(c * DB + (j - 1), 0, D - 1), 0, 0, 0)

    in_specs = [pl.BlockSpec((1, 1, H, W, Cin), sm(j)) for j in range(DB + 2)]
    args = [h] * (DB + 2)
    in_specs += [
        pl.BlockSpec(w2t.shape, lambda n, c: (0, 0, 0)),
        pl.BlockSpec((1, Cout), lambda n, c: (0, 0)),
    ]
    args += [w2t, b2.reshape(1, Cout)]

    return pl.pallas_call(
        _make_conv_body(H, W, Cin, Cout, DB),
        out_shape=jax.ShapeDtypeStruct((N, D, H, W, Cout), out_dtype),
        grid=(N, nc),
        in_specs=in_specs,
        out_specs=pl.BlockSpec((1, DB, H, W, Cout),
                               lambda n, c: (n, c, 0, 0, 0)),
        scratch_shapes=[
            pltpu.VMEM((DB, H * W, Cout), jnp.float32),
        ],
        compiler_params=pltpu.CompilerParams(
            dimension_semantics=("parallel", "parallel"),
            vmem_limit_bytes=_VMEM_LIMIT),
    )(*args)


def kernel(x1, x2, w1, b1, w2, b2):
    N, Cu, Din, Hin, Win = x1.shape
    Cs, Dout, H, W = x2.shape[1], x2.shape[2], x2.shape[3], x2.shape[4]
    Cm = w1.shape[0]
    db = _DB if Dout % _DB == 0 else 1

    x1b = jnp.transpose(x1, (0, 2, 3, 4, 1)).astype(jnp.bfloat16)
    x2b = jnp.transpose(x2, (0, 2, 3, 4, 1)).astype(jnp.bfloat16)
    w1t = jnp.transpose(w1, (2, 3, 4, 1, 0)).reshape(
        27, Cu + Cs, Cm).astype(jnp.bfloat16)
    w2t = jnp.transpose(w2, (2, 3, 4, 1, 0)).reshape(
        27, Cm, w2.shape[0]).astype(jnp.bfloat16)
    mhw = jnp.asarray(np.kron(_interp_mat(Hin, H), _interp_mat(Win, W)),
                      jnp.bfloat16)

    h = _up_conv1(x1b, x2b, w1t, b1, mhw, db)
    y = _conv2(h, w2t, b2, x1.dtype, db)
    return jnp.transpose(y, (0, 4, 1, 2, 3))


# speedup vs baseline: 3.5309x; 1.0053x over previous
import math

import numpy as np

import jax
import jax.numpy as jnp
from jax.experimental import pallas as pl
from jax.experimental.pallas import tpu as pltpu

_VMEM_LIMIT = 64 * 1024 * 1024
_DB = 8


def _interp_mat(n_in, n_out):
    m = np.zeros((n_out, n_in), np.float32)
    for i in range(n_out):
        src = 0.0 if n_out == 1 else i * (n_in - 1) / (n_out - 1)
        i0 = min(int(math.floor(src)), n_in - 1)
        i1 = min(i0 + 1, n_in - 1)
        f = src - i0
        m[i, i0] += 1.0 - f
        m[i, i1] += f
    return m


def _wshift_variants(base, HW, W, C):
    wi = jax.lax.broadcasted_iota(jnp.int32, (HW, 1), 0) % W
    zrow = jnp.zeros((1, C), base.dtype)
    sm1 = jnp.where(wi >= 1,
                    jnp.concatenate([zrow, base[:HW - 1]], axis=0),
                    jnp.zeros_like(base))
    sp1 = jnp.where(wi <= W - 2,
                    jnp.concatenate([base[1:], zrow], axis=0),
                    jnp.zeros_like(base))
    return (sm1, base, sp1)


def _accum_taps(acc_ref, k, variants, w_ref, kd, HW, W):
    for kh in range(3):
        s = (kh - 1) * W
        y = None
        for kw in range(3):
            p = jnp.dot(variants[kw], w_ref[kd * 9 + kh * 3 + kw],
                        preferred_element_type=jnp.float32)
            y = p if y is None else y + p
        if s == 0:
            acc_ref[k] += y
        elif s > 0:
            acc_ref[k, 0:HW - s, :] += y[s:, :]
        else:
            acc_ref[k, -s:HW, :] += y[0:HW + s, :]


def _chunk_conv(acc_ref, var_list, w_ref, c, nc, HW, W, DB):
    for k in range(DB):
        for kd in range(3):
            j = k + kd

            def do(k=k, kd=kd, j=j):
                _accum_taps(acc_ref, k, var_list[j], w_ref, kd, HW, W)

            if k == 0 and kd == 0:
                pl.when(c > 0)(do)
            elif k == DB - 1 and kd == 2:
                pl.when(c < nc - 1)(do)
            else:
                do()


def _make_up_conv_body(Din, Dout, Hin, Win, H, W, Cu, Cs, Cm, DB):
    HW = H * W
    Cin = Cu + Cs

    def body(*refs):
        x1_ref = refs[0]
        x2_refs = refs[1:DB + 3]
        mhw_ref, w_ref, b_ref, o_ref, acc_ref = refs[DB + 3:]
        c = pl.program_id(1)
        nc = pl.num_programs(1)
        acc_ref[...] = jnp.zeros(acc_ref.shape, jnp.float32)

        var_list = []
        for j in range(DB + 2):
            od = jnp.clip(c * DB + (j - 1), 0, Dout - 1)
            t = od * (Din - 1)
            i0 = t // (Dout - 1)
            i1 = jnp.minimum(i0 + 1, Din - 1)
            fd = (t % (Dout - 1)).astype(jnp.float32) * (1.0 / (Dout - 1))
            a0 = x1_ref[0, pl.ds(i0, 1)].reshape(Hin * Win, Cu)
            a1 = x1_ref[0, pl.ds(i1, 1)].reshape(Hin * Win, Cu)
            xz = (a0.astype(jnp.float32)
                  + fd * (a1.astype(jnp.float32) - a0.astype(jnp.float32)))
            u = jnp.dot(mhw_ref[...], xz.astype(jnp.bfloat16),
                        preferred_element_type=jnp.float32)
            base = jnp.concatenate(
                [u.astype(jnp.bfloat16), x2_refs[j][...].reshape(HW, Cs)],
                axis=1)
            var_list.append(_wshift_variants(base, HW, W, Cin))

        _chunk_conv(acc_ref, var_list, w_ref, c, nc, HW, W, DB)

        y = jnp.maximum(acc_ref[...] + b_ref[...], 0.0)
        o_ref[...] = y.reshape(1, DB, H, W, Cm).astype(o_ref.dtype)

    return body


def _up_conv1(x1b, x2b, w1t, b1, mhw, DB):
    N, Din, Hin, Win, Cu = x1b.shape
    _, Dout, H, W, Cs = x2b.shape
    Cm = w1t.shape[-1]
    nc = Dout // DB

    def sm(j):
        return lambda n, c: (n, jnp.clip(c * DB + (j - 1), 0, Dout - 1),
                             0, 0, 0)

    in_specs = [pl.BlockSpec((1, Din, Hin, Win, Cu),
                             lambda n, c: (n, 0, 0, 0, 0))]
    args = [x1b]
    for j in range(DB + 2):
        in_specs.append(pl.BlockSpec((1, 1, H, W, Cs), sm(j)))
        args.append(x2b)
    in_specs += [
        pl.BlockSpec((H * W, Hin * Win), lambda n, c: (0, 0)),
        pl.BlockSpec(w1t.shape, lambda n, c: (0, 0, 0)),
        pl.BlockSpec((1, Cm), lambda n, c: (0, 0)),
    ]
    args += [mhw, w1t, b1.reshape(1, Cm)]

    return pl.pallas_call(
        _make_up_conv_body(Din, Dout, Hin, Win, H, W, Cu, Cs, Cm, DB),
        out_shape=jax.ShapeDtypeStruct((N, Dout, H, W, Cm), jnp.bfloat16),
        grid=(N, nc),
        in_specs=in_specs,
        out_specs=pl.BlockSpec((1, DB, H, W, Cm),
                               lambda n, c: (n, c, 0, 0, 0)),
        scratch_shapes=[
            pltpu.VMEM((DB, H * W, Cm), jnp.float32),
        ],
        compiler_params=pltpu.CompilerParams(
            dimension_semantics=("parallel", "parallel"),
            vmem_limit_bytes=_VMEM_LIMIT),
    )(*args)


def _make_conv_body(H, W, Cin, Cout, DB):
    HW = H * W

    def body(*refs):
        x_refs = refs[:DB + 2]
        w_ref, b_ref, o_ref, acc_ref = refs[DB + 2:]
        c = pl.program_id(1)
        nc = pl.num_programs(1)
        acc_ref[...] = jnp.zeros(acc_ref.shape, jnp.float32)

        var_list = [
            _wshift_variants(x_refs[j][...].reshape(HW, Cin), HW, W, Cin)
            for j in range(DB + 2)]

        _chunk_conv(acc_ref, var_list, w_ref, c, nc, HW, W, DB)

        y = jnp.maximum(acc_ref[...] + b_ref[...], 0.0)
        o_ref[...] = y.reshape(1, DB, H, W, Cout).astype(o_ref.dtype)

    return body


def _conv2(h, w2t, b2, out_dtype, DB):
    N, D, H, W, Cin = h.shape
    Cout = w2t.shape[-1]
    nc = D // DB

    def sm(j):
        return lambda n, c: (n, jnp.clip(c * DB + (j - 1), 0, D - 1), 0, 0, 0)

    in_specs = [pl.BlockSpec((1, 1, H, W, Cin), sm(j)) for j in range(DB + 2)]
    args = [h] * (DB + 2)
    in_specs += [
        pl.BlockSpec(w2t.shape, lambda n, c: (0, 0, 0)),
        pl.BlockSpec((1, Cout), lambda n, c: (0, 0)),
    ]
    args += [w2t, b2.reshape(1, Cout)]

    return pl.pallas_call(
        _make_conv_body(H, W, Cin, Cout, DB),
        out_shape=jax.ShapeDtypeStruct((N, D, H, W, Cout), out_dtype),
        grid=(N, nc),
        in_specs=in_specs,
        out_specs=pl.BlockSpec((1, DB, H, W, Cout),
                               lambda n, c: (n, c, 0, 0, 0)),
        scratch_shapes=[
            pltpu.VMEM((DB, H * W, Cout), jnp.float32),
        ],
        compiler_params=pltpu.CompilerParams(
            dimension_semantics=("parallel", "parallel"),
            vmem_limit_bytes=_VMEM_LIMIT),
    )(*args)


def kernel(x1, x2, w1, b1, w2, b2):
    N, Cu, Din, Hin, Win = x1.shape
    Cs, Dout, H, W = x2.shape[1], x2.shape[2], x2.shape[3], x2.shape[4]
    Cm = w1.shape[0]
    db = _DB if Dout % _DB == 0 else 1

    x1b = jnp.transpose(x1, (0, 2, 3, 4, 1)).astype(jnp.bfloat16)
    x2b = jnp.transpose(x2, (0, 2, 3, 4, 1)).astype(jnp.bfloat16)
    w1t = jnp.transpose(w1, (2, 3, 4, 1, 0)).reshape(
        27, Cu + Cs, Cm).astype(jnp.bfloat16)
    w2t = jnp.transpose(w2, (2, 3, 4, 1, 0)).reshape(
        27, Cm, w2.shape[0]).astype(jnp.bfloat16)
    mhw = jnp.asarray(np.kron(_interp_mat(Hin, H), _interp_mat(Win, W)),
                      jnp.bfloat16)

    h = _up_conv1(x1b, x2b, w1t, b1, mhw, db)
    y = _conv2(h, w2t, b2, x1.dtype, db)
    return jnp.transpose(y, (0, 4, 1, 2, 3))


# speedup vs baseline: 3.6250x; 1.0267x over previous
import math

import numpy as np

import jax
import jax.numpy as jnp
from jax.experimental import pallas as pl
from jax.experimental.pallas import tpu as pltpu

_VMEM_LIMIT = 64 * 1024 * 1024
_DB = 8


def _interp_mat(n_in, n_out):
    m = np.zeros((n_out, n_in), np.float32)
    for i in range(n_out):
        src = 0.0 if n_out == 1 else i * (n_in - 1) / (n_out - 1)
        i0 = min(int(math.floor(src)), n_in - 1)
        i1 = min(i0 + 1, n_in - 1)
        f = src - i0
        m[i, i0] += 1.0 - f
        m[i, i1] += f
    return m


def _wshift_variants(base, HW, W, C):
    wi = jax.lax.broadcasted_iota(jnp.int32, (HW, 1), 0) % W
    zrow = jnp.zeros((1, C), base.dtype)
    sm1 = jnp.where(wi >= 1,
                    jnp.concatenate([zrow, base[:HW - 1]], axis=0),
                    jnp.zeros_like(base))
    sp1 = jnp.where(wi <= W - 2,
                    jnp.concatenate([base[1:], zrow], axis=0),
                    jnp.zeros_like(base))
    return (sm1, base, sp1)


def _accum_taps(acc_ref, k, variants, w_ref, kd, HW, W):
    for kh in range(3):
        s = (kh - 1) * W
        y = None
        for kw in range(3):
            p = jnp.dot(variants[kw], w_ref[kd * 9 + kh * 3 + kw],
                        preferred_element_type=jnp.float32)
            y = p if y is None else y + p
        if s == 0:
            acc_ref[k] += y
        elif s > 0:
            acc_ref[k, 0:HW - s, :] += y[s:, :]
        else:
            acc_ref[k, -s:HW, :] += y[0:HW + s, :]


def _chunk_conv(acc_ref, var_list, w_ref, c, nc, HW, W, DB):
    for k in range(DB):
        for kd in range(3):
            j = k + kd

            def do(k=k, kd=kd, j=j):
                _accum_taps(acc_ref, k, var_list[j], w_ref, kd, HW, W)

            if k == 0 and kd == 0:
                pl.when(c > 0)(do)
            elif k == DB - 1 and kd == 2:
                pl.when(c < nc - 1)(do)
            else:
                do()


def _make_up_conv_body(Din, Dout, Hin, Win, H, W, Cu, Cs, Cm, DB):
    HW = H * W
    Cin = Cu + Cs

    def body(*refs):
        x1_ref = refs[0]
        x2_refs = refs[1:DB + 3]
        mhw_ref, w_ref, b_ref, o_ref, acc_ref = refs[DB + 3:]
        c = pl.program_id(1)
        nc = pl.num_programs(1)
        acc_ref[...] = jnp.zeros(acc_ref.shape, jnp.float32)

        var_list = []
        for j in range(DB + 2):
            od = jnp.clip(c * DB + (j - 1), 0, Dout - 1)
            t = od * (Din - 1)
            i0 = t // (Dout - 1)
            i1 = jnp.minimum(i0 + 1, Din - 1)
            fd = (t % (Dout - 1)).astype(jnp.float32) * (1.0 / (Dout - 1))
            a0 = x1_ref[0, pl.ds(i0, 1)].reshape(Hin * Win, Cu)
            a1 = x1_ref[0, pl.ds(i1, 1)].reshape(Hin * Win, Cu)
            xz = a0 + fd.astype(jnp.bfloat16) * (a1 - a0)
            u = jnp.dot(mhw_ref[...], xz,
                        preferred_element_type=jnp.float32)
            base = jnp.concatenate(
                [u.astype(jnp.bfloat16), x2_refs[j][...].reshape(HW, Cs)],
                axis=1)
            var_list.append(_wshift_variants(base, HW, W, Cin))

        _chunk_conv(acc_ref, var_list, w_ref, c, nc, HW, W, DB)

        y = jnp.maximum(acc_ref[...] + b_ref[...], 0.0)
        o_ref[...] = y.reshape(1, DB, H, W, Cm).astype(o_ref.dtype)

    return body


def _up_conv1(x1b, x2b, w1t, b1, mhw, DB):
    N, Din, Hin, Win, Cu = x1b.shape
    _, Dout, H, W, Cs = x2b.shape
    Cm = w1t.shape[-1]
    nc = Dout // DB

    def sm(j):
        return lambda n, c: (n, jnp.clip(c * DB + (j - 1), 0, Dout - 1),
                             0, 0, 0)

    in_specs = [pl.BlockSpec((1, Din, Hin, Win, Cu),
                             lambda n, c: (n, 0, 0, 0, 0))]
    args = [x1b]
    for j in range(DB + 2):
        in_specs.append(pl.BlockSpec((1, 1, H, W, Cs), sm(j)))
        args.append(x2b)
    in_specs += [
        pl.BlockSpec((H * W, Hin * Win), lambda n, c: (0, 0)),
        pl.BlockSpec(w1t.shape, lambda n, c: (0, 0, 0)),
        pl.BlockSpec((1, Cm), lambda n, c: (0, 0)),
    ]
    args += [mhw, w1t, b1.reshape(1, Cm)]

    return pl.pallas_call(
        _make_up_conv_body(Din, Dout, Hin, Win, H, W, Cu, Cs, Cm, DB),
        out_shape=jax.ShapeDtypeStruct((N, Dout, H, W, Cm), jnp.bfloat16),
        grid=(N, nc),
        in_specs=in_specs,
        out_specs=pl.BlockSpec((1, DB, H, W, Cm),
                               lambda n, c: (n, c, 0, 0, 0)),
        scratch_shapes=[
            pltpu.VMEM((DB, H * W, Cm), jnp.float32),
        ],
        compiler_params=pltpu.CompilerParams(
            dimension_semantics=("parallel", "parallel"),
            vmem_limit_bytes=_VMEM_LIMIT),
    )(*args)


def _make_conv_body(H, W, Cin, Cout, DB):
    HW = H * W

    def body(*refs):
        x_refs = refs[:DB + 2]
        w_ref, b_ref, o_ref, acc_ref = refs[DB + 2:]
        c = pl.program_id(1)
        nc = pl.num_programs(1)
        acc_ref[...] = jnp.zeros(acc_ref.shape, jnp.float32)

        var_list = [
            _wshift_variants(x_refs[j][...].reshape(HW, Cin), HW, W, Cin)
            for j in range(DB + 2)]

        _chunk_conv(acc_ref, var_list, w_ref, c, nc, HW, W, DB)

        y = jnp.maximum(acc_ref[...] + b_ref[...], 0.0)
        o_ref[...] = y.reshape(1, DB, H, W, Cout).astype(o_ref.dtype)

    return body


def _conv2(h, w2t, b2, out_dtype, DB):
    N, D, H, W, Cin = h.shape
    Cout = w2t.shape[-1]
    nc = D // DB

    def sm(j):
        return lambda n, c: (n, jnp.clip(c * DB + (j - 1), 0, D - 1), 0, 0, 0)

    in_specs = [pl.BlockSpec((1, 1, H, W, Cin), sm(j)) for j in range(DB + 2)]
    args = [h] * (DB + 2)
    in_specs += [
        pl.BlockSpec(w2t.shape, lambda n, c: (0, 0, 0)),
        pl.BlockSpec((1, Cout), lambda n, c: (0, 0)),
    ]
    args += [w2t, b2.reshape(1, Cout)]

    return pl.pallas_call(
        _make_conv_body(H, W, Cin, Cout, DB),
        out_shape=jax.ShapeDtypeStruct((N, D, H, W, Cout), out_dtype),
        grid=(N, nc),
        in_specs=in_specs,
        out_specs=pl.BlockSpec((1, DB, H, W, Cout),
                               lambda n, c: (n, c, 0, 0, 0)),
        scratch_shapes=[
            pltpu.VMEM((DB, H * W, Cout), jnp.float32),
        ],
        compiler_params=pltpu.CompilerParams(
            dimension_semantics=("parallel", "parallel"),
            vmem_limit_bytes=_VMEM_LIMIT),
    )(*args)


def kernel(x1, x2, w1, b1, w2, b2):
    N, Cu, Din, Hin, Win = x1.shape
    Cs, Dout, H, W = x2.shape[1], x2.shape[2], x2.shape[3], x2.shape[4]
    Cm = w1.shape[0]
    db = _DB if Dout % _DB == 0 else 1

    x1b = jnp.transpose(x1, (0, 2, 3, 4, 1)).astype(jnp.bfloat16)
    x2b = jnp.transpose(x2, (0, 2, 3, 4, 1)).astype(jnp.bfloat16)
    w1t = jnp.transpose(w1, (2, 3, 4, 1, 0)).reshape(
        27, Cu + Cs, Cm).astype(jnp.bfloat16)
    w2t = jnp.transpose(w2, (2, 3, 4, 1, 0)).reshape(
        27, Cm, w2.shape[0]).astype(jnp.bfloat16)
    mhw = jnp.asarray(np.kron(_interp_mat(Hin, H), _interp_mat(Win, W)),
                      jnp.bfloat16)

    h = _up_conv1(x1b, x2b, w1t, b1, mhw, db)
    y = _conv2(h, w2t, b2, x1.dtype, db)
    return jnp.transpose(y, (0, 4, 1, 2, 3))
